# baseline (speedup 1.0000x reference)
import functools

import jax
import jax.numpy as jnp
from jax import lax
from jax.experimental import pallas as pl
from jax.experimental.pallas import tpu as pltpu
from jax.experimental.pallas import tpu_sc as plsc

C = 128
K = 80
N_POINTS = 100000
N_CHUNKS = N_POINTS // K
NW = 32
MAXCH = 40
IDX_ROWS = MAXCH * NW
G32 = C // 32
JGROUPS = C // 16
HIGH_MASK = -65536


def _sc_loss(imgt, idx2d, feature):
    mesh = plsc.VectorSubcoreMesh(core_axis_name="c", subcore_axis_name="s")

    @functools.partial(
        pl.kernel,
        mesh=mesh,
        compiler_params=pltpu.CompilerParams(needs_layout_passes=False,
                                             use_tc_tiling_on_sc=False),
        out_type=jax.ShapeDtypeStruct((NW, JGROUPS, 16), jnp.float32),
        scratch_types=[
            pltpu.VMEM((MAXCH, K), jnp.int32),
            pltpu.VMEM((K, C // 2), jnp.int32),
            pltpu.VMEM((K, C // 2), jnp.int32),
            pltpu.VMEM((K, C // 2), jnp.int32),
            pltpu.VMEM((K, C // 2), jnp.int32),
            pltpu.VMEM((JGROUPS, 16), jnp.float32),
            pltpu.SemaphoreType.DMA,
            pltpu.SemaphoreType.DMA,
            pltpu.SemaphoreType.DMA,
            pltpu.SemaphoreType.DMA,
        ],
    )
    def k(imgt_hbm, idx_hbm, feat_hbm, out_hbm, idx_v, img0_v, img1_v,
          feat0_v, feat1_v, acc_v, sg0, sg1, sf0, sf1):
        wid = lax.axis_index("s") * 2 + lax.axis_index("c")
        base_ch = MAXCH * wid
        nch = jnp.minimum(MAXCH, jnp.maximum(N_CHUNKS - base_ch, 0))

        pltpu.sync_copy(idx_hbm.at[pl.ds(base_ch, MAXCH)], idx_v)

        for j in range(JGROUPS):
            acc_v[j, :] = jnp.zeros((16,), jnp.float32)

        bufs = ((img0_v, feat0_v, sg0, sf0), (img1_v, feat1_v, sg1, sf1))

        def issue(ci, b):
            img_b, feat_b, sg, sf = bufs[b]

            @pl.when(ci < nch)
            def _():
                pltpu.async_copy(imgt_hbm.at[idx_v.at[ci]], img_b, sg)
                pltpu.async_copy(feat_hbm.at[pl.ds((base_ch + ci) * K, K)],
                                 feat_b, sf)

        def consume(ci, b):
            img_b, feat_b, sg, sf = bufs[b]

            @pl.when(ci < nch)
            def _():
                pltpu.make_async_copy(imgt_hbm.at[idx_v.at[ci]], img_b,
                                      sg).wait()
                pltpu.make_async_copy(
                    feat_hbm.at[pl.ds((base_ch + ci) * K, K)], feat_b,
                    sf).wait()

                def row_body(r, accs):
                    new = list(accs)
                    for j in range(G32):
                        gi = img_b[r, pl.ds(j * 16, 16)]
                        ti = feat_b[r, pl.ds(j * 16, 16)]
                        g0 = plsc.bitcast(gi << 16, jnp.float32)
                        g1 = plsc.bitcast(gi & HIGH_MASK, jnp.float32)
                        t0 = plsc.bitcast(ti << 16, jnp.float32)
                        t1 = plsc.bitcast(ti & HIGH_MASK, jnp.float32)
                        d0 = g0 * 200.0 - t0
                        d1 = g1 * 200.0 - t1
                        new[2 * j] = new[2 * j] + d0 * d0
                        new[2 * j + 1] = new[2 * j + 1] + d1 * d1
                    return tuple(new)

                accs = lax.fori_loop(
                    0, K, row_body,
                    tuple(acc_v[j, :] for j in range(JGROUPS)))
                for j in range(JGROUPS):
                    acc_v[j, :] = accs[j]

        issue(0, 0)
        issue(1, 1)

        def outer(ci, _):
            consume(ci, 0)
            issue(ci + 2, 0)
            consume(ci + 1, 1)
            issue(ci + 3, 1)
            return 0

        lax.fori_loop(0, MAXCH // 2, lambda i, c: outer(2 * i, c), 0)

        for j in range(JGROUPS):
            acc_v[j, :] = acc_v[j, :] * (1.0 / 40000.0)
        pltpu.sync_copy(acc_v, out_hbm.at[wid])

    return k(imgt, idx2d, feature)


def kernel(image, points, feature):
    imgt = lax.bitcast_convert_type(
        image[0].reshape(C, -1).T.astype(jnp.bfloat16).reshape(-1, C // 2, 2),
        jnp.int32)
    featb = lax.bitcast_convert_type(
        feature.astype(jnp.bfloat16).reshape(-1, C // 2, 2), jnp.int32)
    idx = points[:, 0] * 4096 + points[:, 1] * 64 + points[:, 2]
    idx2d = jnp.zeros((IDX_ROWS * K,), jnp.int32).at[:N_POINTS].set(
        idx.astype(jnp.int32)).reshape(IDX_ROWS, K)
    partials = _sc_loss(imgt, idx2d, featb)
    return jnp.sum(partials)


# speedup vs baseline: 13.6226x; 13.6226x over previous
import functools

import jax
import jax.numpy as jnp
from jax import lax
from jax.experimental import pallas as pl
from jax.experimental.pallas import tpu as pltpu
from jax.experimental.pallas import tpu_sc as plsc

C = 128
K = 80
N_POINTS = 100000
N_CHUNKS = N_POINTS // K
NW = 32
MAXCH = 40
IDX_ROWS = MAXCH * NW
JGROUPS = C // 16


def _sc_loss(imgt, idx2d, feature):
    mesh = plsc.VectorSubcoreMesh(core_axis_name="c", subcore_axis_name="s")

    @functools.partial(
        pl.kernel,
        mesh=mesh,
        out_type=jax.ShapeDtypeStruct((NW, JGROUPS, 2, 16), jnp.float32),
        scratch_types=[
            pltpu.VMEM((MAXCH, K), jnp.int32),
            pltpu.VMEM((K, C), jnp.float32),
            pltpu.VMEM((K, C), jnp.float32),
            pltpu.VMEM((K, C), jnp.bfloat16),
            pltpu.VMEM((K, C), jnp.bfloat16),
            pltpu.VMEM((JGROUPS, 2, 16), jnp.float32),
            pltpu.SemaphoreType.DMA,
            pltpu.SemaphoreType.DMA,
            pltpu.SemaphoreType.DMA,
            pltpu.SemaphoreType.DMA,
        ],
    )
    def k(imgt_hbm, idx_hbm, feat_hbm, out_hbm, idx_v, img0_v, img1_v,
          feat0_v, feat1_v, acc_v, sg0, sg1, sf0, sf1):
        wid = lax.axis_index("s") * 2 + lax.axis_index("c")
        base_ch = MAXCH * wid
        nch = jnp.minimum(MAXCH, jnp.maximum(N_CHUNKS - base_ch, 0))

        pltpu.sync_copy(idx_hbm.at[pl.ds(base_ch, MAXCH)], idx_v)

        for j in range(JGROUPS):
            acc_v[j, :, :] = jnp.zeros((2, 16), jnp.float32)

        bufs = ((img0_v, feat0_v, sg0, sf0), (img1_v, feat1_v, sg1, sf1))

        def issue(ci, b):
            img_b, feat_b, sg, sf = bufs[b]

            @pl.when(ci < nch)
            def _():
                pltpu.async_copy(imgt_hbm.at[idx_v.at[ci]], img_b, sg)
                pltpu.async_copy(feat_hbm.at[pl.ds((base_ch + ci) * K, K)],
                                 feat_b, sf)

        def consume(ci, b):
            img_b, feat_b, sg, sf = bufs[b]

            @pl.when(ci < nch)
            def _():
                pltpu.make_async_copy(imgt_hbm.at[idx_v.at[ci]], img_b,
                                      sg).wait()
                pltpu.make_async_copy(
                    feat_hbm.at[pl.ds((base_ch + ci) * K, K)], feat_b,
                    sf).wait()

                def row_body(r, accs):
                    new = list(accs)
                    rr = pl.multiple_of(2 * r, 2)
                    for j in range(JGROUPS):
                        g2 = img_b[pl.ds(rr, 2), pl.ds(j * 16, 16)]
                        t2 = feat_b[pl.ds(rr, 2), pl.ds(j * 16, 16)]
                        d = g2 * 200.0 - t2.astype(jnp.float32)
                        new[j] = new[j] + d * d
                    return tuple(new)

                accs = lax.fori_loop(
                    0, K // 2, row_body,
                    tuple(acc_v[j, :, :] for j in range(JGROUPS)))
                for j in range(JGROUPS):
                    acc_v[j, :, :] = accs[j]

        issue(0, 0)
        issue(1, 1)

        def outer(ci, _):
            consume(ci, 0)
            issue(ci + 2, 0)
            consume(ci + 1, 1)
            issue(ci + 3, 1)
            return 0

        lax.fori_loop(0, MAXCH // 2, lambda i, c: outer(2 * i, c), 0)

        for j in range(JGROUPS):
            acc_v[j, :, :] = acc_v[j, :, :] * (1.0 / 40000.0)
        pltpu.sync_copy(acc_v, out_hbm.at[wid])

    return k(imgt, idx2d, feature)


def kernel(image, points, feature):
    imgt = image[0].reshape(C, -1).T
    featb = feature.astype(jnp.bfloat16)
    idx = points[:, 0] * 4096 + points[:, 1] * 64 + points[:, 2]
    idx2d = jnp.zeros((IDX_ROWS * K,), jnp.int32).at[:N_POINTS].set(
        idx.astype(jnp.int32)).reshape(IDX_ROWS, K)
    partials = _sc_loss(imgt, idx2d, featb)
    return jnp.sum(partials)


# speedup vs baseline: 16.9240x; 1.2423x over previous
import functools

import jax
import jax.numpy as jnp
from jax import lax
from jax.experimental import pallas as pl
from jax.experimental.pallas import tpu as pltpu
from jax.experimental.pallas import tpu_sc as plsc

C = 128
K = 80
N_POINTS = 100000
N_CHUNKS = N_POINTS // K
NW = 32
MAXCH = 40
IDX_ROWS = MAXCH * NW
JGROUPS = C // 16


def _sc_loss(imgt, idx2d, feature):
    mesh = plsc.VectorSubcoreMesh(core_axis_name="c", subcore_axis_name="s")

    @functools.partial(
        pl.kernel,
        mesh=mesh,
        out_type=jax.ShapeDtypeStruct((NW, 2, C), jnp.float32),
        scratch_types=[
            pltpu.VMEM((MAXCH, K), jnp.int32),
            pltpu.VMEM((K, C), jnp.float32),
            pltpu.VMEM((K, C), jnp.float32),
            pltpu.VMEM((K, C), jnp.float32),
            pltpu.VMEM((K, C), jnp.float32),
            pltpu.VMEM((2, C), jnp.float32),
            pltpu.SemaphoreType.DMA,
            pltpu.SemaphoreType.DMA,
            pltpu.SemaphoreType.DMA,
            pltpu.SemaphoreType.DMA,
        ],
    )
    def k(imgt_hbm, idx_hbm, feat_hbm, out_hbm, idx_v, img0_v, img1_v,
          feat0_v, feat1_v, acc_v, sg0, sg1, sf0, sf1):
        wid = lax.axis_index("s") * 2 + lax.axis_index("c")
        base_ch = MAXCH * wid
        nch = jnp.minimum(MAXCH, jnp.maximum(N_CHUNKS - base_ch, 0))

        pltpu.sync_copy(idx_hbm.at[pl.ds(base_ch, MAXCH)], idx_v)

        acc_v[:, :] = jnp.zeros((2, C), jnp.float32)

        bufs = ((img0_v, feat0_v, sg0, sf0), (img1_v, feat1_v, sg1, sf1))

        def issue(ci, b):
            img_b, feat_b, sg, sf = bufs[b]

            @pl.when(ci < nch)
            def _():
                pltpu.async_copy(imgt_hbm.at[idx_v.at[ci]], img_b, sg)
                pltpu.async_copy(feat_hbm.at[pl.ds((base_ch + ci) * K, K)],
                                 feat_b, sf)

        def consume(ci, b):
            img_b, feat_b, sg, sf = bufs[b]

            @pl.when(ci < nch)
            def _():
                pltpu.make_async_copy(imgt_hbm.at[idx_v.at[ci]], img_b,
                                      sg).wait()
                pltpu.make_async_copy(
                    feat_hbm.at[pl.ds((base_ch + ci) * K, K)], feat_b,
                    sf).wait()

                def row_body(r, acc):
                    rr = pl.multiple_of(2 * r, 2)
                    g2 = img_b[pl.ds(rr, 2), :]
                    t2 = feat_b[pl.ds(rr, 2), :]
                    d = g2 * 200.0 - t2
                    return acc + d * d

                acc_v[:, :] = lax.fori_loop(0, K // 2, row_body,
                                            acc_v[:, :])

        issue(0, 0)
        issue(1, 1)

        def outer(ci, _):
            consume(ci, 0)
            issue(ci + 2, 0)
            consume(ci + 1, 1)
            issue(ci + 3, 1)
            return 0

        lax.fori_loop(0, MAXCH // 2, lambda i, c: outer(2 * i, c), 0)

        acc_v[:, :] = acc_v[:, :] * (1.0 / 40000.0)
        pltpu.sync_copy(acc_v, out_hbm.at[wid])

    return k(imgt, idx2d, feature)


def kernel(image, points, feature):
    imgt = image[0].reshape(C, -1).T
    idx = points[:, 0] * 4096 + points[:, 1] * 64 + points[:, 2]
    idx2d = jnp.zeros((IDX_ROWS * K,), jnp.int32).at[:N_POINTS].set(
        idx.astype(jnp.int32)).reshape(IDX_ROWS, K)
    partials = _sc_loss(imgt, idx2d, feature)
    return jnp.sum(partials)


# speedup vs baseline: 20.2186x; 1.1947x over previous
import functools

import jax
import jax.numpy as jnp
from jax import lax
from jax.experimental import pallas as pl
from jax.experimental.pallas import tpu as pltpu
from jax.experimental.pallas import tpu_sc as plsc

C = 128
K = 80
N_POINTS = 100000
N_CHUNKS = N_POINTS // K
NW = 32
MAXCH = 40
IDX_ROWS = MAXCH * NW
JGROUPS = C // 16


def _sc_loss(imgt, idx2d, feature):
    mesh = plsc.VectorSubcoreMesh(core_axis_name="c", subcore_axis_name="s")

    @functools.partial(
        pl.kernel,
        mesh=mesh,
        out_type=jax.ShapeDtypeStruct((NW, 2, C), jnp.float32),
        scratch_types=[
            pltpu.VMEM((MAXCH, K), jnp.int32),
        ] + [pltpu.VMEM((K, C), jnp.float32)] * 8
          + [pltpu.VMEM((2, C), jnp.float32)]
          + [pltpu.SemaphoreType.DMA] * 8,
    )
    def k(imgt_hbm, idx_hbm, feat_hbm, out_hbm, idx_v,
          img0_v, img1_v, img2_v, img3_v,
          feat0_v, feat1_v, feat2_v, feat3_v, acc_v,
          sg0, sg1, sg2, sg3, sf0, sf1, sf2, sf3):
        wid = lax.axis_index("s") * 2 + lax.axis_index("c")
        base_ch = MAXCH * wid
        nch = jnp.minimum(MAXCH, jnp.maximum(N_CHUNKS - base_ch, 0))

        pltpu.sync_copy(idx_hbm.at[pl.ds(base_ch, MAXCH)], idx_v)

        acc_v[:, :] = jnp.zeros((2, C), jnp.float32)

        bufs = ((img0_v, feat0_v, sg0, sf0), (img1_v, feat1_v, sg1, sf1),
                (img2_v, feat2_v, sg2, sf2), (img3_v, feat3_v, sg3, sf3))

        def issue(ci, b):
            img_b, feat_b, sg, sf = bufs[b]

            @pl.when(ci < nch)
            def _():
                pltpu.async_copy(imgt_hbm.at[idx_v.at[ci]], img_b, sg)
                pltpu.async_copy(feat_hbm.at[pl.ds((base_ch + ci) * K, K)],
                                 feat_b, sf)

        def consume(ci, b):
            img_b, feat_b, sg, sf = bufs[b]

            @pl.when(ci < nch)
            def _():
                pltpu.make_async_copy(imgt_hbm.at[idx_v.at[ci]], img_b,
                                      sg).wait()
                pltpu.make_async_copy(
                    feat_hbm.at[pl.ds((base_ch + ci) * K, K)], feat_b,
                    sf).wait()

                def row_body(r, acc):
                    rr = pl.multiple_of(2 * r, 2)
                    g2 = img_b[pl.ds(rr, 2), :]
                    t2 = feat_b[pl.ds(rr, 2), :]
                    d = g2 * 200.0 - t2
                    return acc + d * d

                acc_v[:, :] = lax.fori_loop(0, K // 2, row_body,
                                            acc_v[:, :])

        for b in range(4):
            issue(b, b)

        def outer(ci, _):
            for b in range(4):
                consume(ci + b, b)
                issue(ci + 4 + b, b)
            return 0

        lax.fori_loop(0, MAXCH // 4, lambda i, c: outer(4 * i, c), 0)

        acc_v[:, :] = acc_v[:, :] * (1.0 / 40000.0)
        pltpu.sync_copy(acc_v, out_hbm.at[wid])

    return k(imgt, idx2d, feature)


def kernel(image, points, feature):
    imgt = image[0].reshape(C, -1).T
    idx = points[:, 0] * 4096 + points[:, 1] * 64 + points[:, 2]
    idx2d = jnp.zeros((IDX_ROWS * K,), jnp.int32).at[:N_POINTS].set(
        idx.astype(jnp.int32)).reshape(IDX_ROWS, K)
    partials = _sc_loss(imgt, idx2d, feature)
    return jnp.sum(partials)
